# baseline (speedup 1.0000x reference)
import functools

import jax
import jax.numpy as jnp
from jax import lax
from jax.experimental import pallas as pl
from jax.experimental.pallas import tpu as pltpu
from jax.experimental.pallas import tpu_sc as plsc

DIM = 128
EPS = 1e-12
NUM_WORKERS = 32
CHUNK = 128


def _sc_token_gather(x2d, token_table):
    n = x2d.shape[0] * x2d.shape[1] * CHUNK
    per_w = n // NUM_WORKERS
    n_chunks = per_w // CHUNK
    mesh = plsc.VectorSubcoreMesh(core_axis_name="c", subcore_axis_name="s")

    @functools.partial(
        pl.kernel,
        mesh=mesh,
        out_type=jax.ShapeDtypeStruct((n, DIM), jnp.float32),
        scratch_types=[
            pltpu.VMEM((n_chunks, CHUNK), jnp.int32),
            pltpu.VMEM((CHUNK, DIM), jnp.float32),
            pltpu.VMEM((CHUNK, DIM), jnp.float32),
            pltpu.SemaphoreType.DMA,
            pltpu.SemaphoreType.DMA,
            pltpu.SemaphoreType.DMA,
            pltpu.SemaphoreType.DMA,
        ],
    )
    def k(x_ref, tab_ref, out_ref, idx_all, r0, r1, gsem0, gsem1, wsem0,
          wsem1):
        num_cores = 2
        wid = lax.axis_index("s") * num_cores + lax.axis_index("c")
        base_w = wid * per_w

        def g_start(c, buf, sem):
            pltpu.async_copy(tab_ref.at[idx_all.at[c]], buf, sem)

        def g_wait(c, buf, sem):
            pltpu.make_async_copy(tab_ref.at[idx_all.at[c]], buf, sem).wait()

        def w_start(c, buf, sem):
            pltpu.async_copy(buf, out_ref.at[pl.ds(base_w + c * CHUNK, CHUNK)],
                             sem)

        def w_wait(c, buf, sem):
            pltpu.make_async_copy(
                buf, out_ref.at[pl.ds(base_w + c * CHUNK, CHUNK)], sem).wait()

        pltpu.sync_copy(x_ref.at[wid], idx_all)
        g_start(0, r0, gsem0)
        g_start(1, r1, gsem1)
        g_wait(0, r0, gsem0)
        w_start(0, r0, wsem0)

        def body(g, carry):
            c = 2 * g
            w_wait(c - 2, r0, wsem0)
            g_start(c, r0, gsem0)
            g_wait(c - 1, r1, gsem1)
            w_start(c - 1, r1, wsem1)
            w_wait(c - 1, r1, wsem1)
            g_start(c + 1, r1, gsem1)
            g_wait(c, r0, gsem0)
            w_start(c, r0, wsem0)
            return carry

        lax.fori_loop(1, n_chunks // 2, body, 0)
        g_wait(n_chunks - 1, r1, gsem1)
        w_start(n_chunks - 1, r1, wsem1)
        w_wait(n_chunks - 2, r0, wsem0)
        w_wait(n_chunks - 1, r1, wsem1)

    return k(x2d, token_table)


def _gather16(vec, idx):
    dnums = lax.GatherDimensionNumbers(
        offset_dims=(), collapsed_slice_dims=(0,), start_index_map=(0,))
    return lax.gather(vec, idx.reshape(16, 1), dnums, (1,),
                      mode=lax.GatherScatterMode.PROMISE_IN_BOUNDS)


def _splat(vec, lane):
    return _gather16(vec, jnp.full((16,), lane, dtype=jnp.int32))


def _rsqrt_newton(x):
    i = lax.bitcast_convert_type(x, jnp.int32).astype(jnp.float32)
    i = (jnp.float32(1597463007.0) - 0.5 * i).astype(jnp.int32)
    y = lax.bitcast_convert_type(i, jnp.float32)
    for _ in range(3):
        y = y * (1.5 - 0.5 * x * y * y)
    return y


def _lanesum(v):
    lanes = lax.iota(jnp.int32, 16)
    for k in (1, 2, 4, 8):
        v = v + _gather16(v, lax.bitwise_xor(lanes, jnp.int32(k)))
    return v


def _sc_fused(x3d, tt3d, posb, dt, gamma, beta, token_table, seq_len):
    n = x3d.shape[0] * x3d.shape[1] * CHUNK
    per_w = n // NUM_WORKERS
    n_chunks = per_w // CHUNK
    n_main = (n_chunks // 3) * 3
    mesh = plsc.VectorSubcoreMesh(core_axis_name="c", subcore_axis_name="s")
    nvec = DIM // 16

    @functools.partial(
        pl.kernel,
        mesh=mesh,
        out_type=jax.ShapeDtypeStruct((n, DIM), jnp.float32),
        scratch_types=[
            pltpu.VMEM((n_chunks, CHUNK), jnp.int32),
            pltpu.VMEM((n_chunks, CHUNK), jnp.int32),
            pltpu.VMEM((seq_len, DIM), jnp.float32),
            pltpu.VMEM((DIM,), jnp.float32),
            pltpu.VMEM((DIM,), jnp.float32),
            pltpu.VMEM((DIM,), jnp.float32),
            pltpu.VMEM((CHUNK, DIM), jnp.float32),
            pltpu.VMEM((CHUNK, DIM), jnp.float32),
            pltpu.VMEM((CHUNK, DIM), jnp.float32),
            pltpu.SemaphoreType.DMA,
            pltpu.SemaphoreType.DMA,
            pltpu.SemaphoreType.DMA,
            pltpu.SemaphoreType.DMA,
            pltpu.SemaphoreType.DMA,
            pltpu.SemaphoreType.DMA,
        ],
    )
    def k(x_ref, tt_ref, posb_ref, dt_ref, g_ref, b_ref, tab_ref, out_ref,
          idx_all, tt_all, posb_v, dt_v, g_v, b_v, r0, r1, r2,
          gs0, gs1, gs2, ws0, ws1, ws2):
        num_cores = 2
        wid = lax.axis_index("s") * num_cores + lax.axis_index("c")
        base_w = wid * per_w
        bufs = (r0, r1, r2)
        gsems = (gs0, gs1, gs2)
        wsems = (ws0, ws1, ws2)

        pltpu.sync_copy(x_ref.at[wid], idx_all)
        pltpu.sync_copy(tt_ref.at[wid], tt_all)
        pltpu.sync_copy(posb_ref, posb_v)
        pltpu.sync_copy(dt_ref, dt_v)
        pltpu.sync_copy(g_ref, g_v)
        pltpu.sync_copy(b_ref, b_v)

        dtv = [dt_v[pl.ds(16 * v, 16)] for v in range(nvec)]
        gv = [g_v[pl.ds(16 * v, 16)] for v in range(nvec)]
        bv = [b_v[pl.ds(16 * v, 16)] for v in range(nvec)]
        inv_d = jnp.float32(1.0 / DIM)

        def g_start(c, buf, sem):
            pltpu.async_copy(tab_ref.at[idx_all.at[c]], buf, sem)

        def g_wait(c, buf, sem):
            pltpu.make_async_copy(tab_ref.at[idx_all.at[c]], buf, sem).wait()

        def w_start(c, buf, sem):
            pltpu.async_copy(buf, out_ref.at[pl.ds(base_w + c * CHUNK, CHUNK)],
                             sem)

        def w_wait(c, buf, sem):
            pltpu.make_async_copy(
                buf, out_ref.at[pl.ds(base_w + c * CHUNK, CHUNK)], sem).wait()

        def compute(c, buf):
            base_l = lax.rem(base_w + c * CHUNK, seq_len)

            @plsc.parallel_loop(0, CHUNK, unroll=6)
            def token(j):
                l0 = base_l + j
                l = jnp.where(l0 >= seq_len, l0 - seq_len, l0)
                ttv = tt_all[c, pl.ds((j // 16) * 16, 16)]
                ttf = _splat(ttv, lax.rem(j, 16)).astype(jnp.float32)
                h = [buf[j, pl.ds(16 * v, 16)]
                     + posb_v[l, pl.ds(16 * v, 16)]
                     + ttf * dtv[v] for v in range(nvec)]
                s = h[0]
                q = h[0] * h[0]
                for v in range(1, nvec):
                    s = s + h[v]
                    q = q + h[v] * h[v]
                mean = _lanesum(s) * inv_d
                var = _lanesum(q) * inv_d - mean * mean
                rstd = _rsqrt_newton(var + EPS)
                mb = mean * rstd
                for v in range(nvec):
                    buf[j, pl.ds(16 * v, 16)] = ((h[v] * rstd - mb) * gv[v]
                                                 + bv[v])

        g_start(0, bufs[0], gsems[0])
        g_start(1, bufs[1], gsems[1])

        def group(i, carry):
            for kk in range(3):
                c = 3 * i + kk
                g_wait(c, bufs[kk], gsems[kk])
                compute(c, bufs[kk])
                if kk == 0:
                    @pl.when(i > 0)
                    def _():
                        w_wait(c - 1, bufs[2], wsems[2])
                else:
                    w_wait(c - 1, bufs[kk - 1], wsems[kk - 1])
                g_start(c + 2, bufs[(kk + 2) % 3], gsems[(kk + 2) % 3])
                w_start(c, bufs[kk], wsems[kk])
            return carry

        lax.fori_loop(0, n_main // 3, group, 0)

        g_wait(n_main, bufs[0], gsems[0])
        compute(n_main, bufs[0])
        w_wait(n_main - 1, bufs[2], wsems[2])
        w_start(n_main, bufs[0], wsems[0])
        g_wait(n_main + 1, bufs[1], gsems[1])
        compute(n_main + 1, bufs[1])
        w_start(n_main + 1, bufs[1], wsems[1])
        w_wait(n_main, bufs[0], wsems[0])
        w_wait(n_main + 1, bufs[1], wsems[1])

    return k(x3d, tt3d, posb, dt, gamma, beta, token_table)


def _tc_ln(h, ttf, pos, type_table, gamma, beta, seq_len):
    n = h.shape[0]
    rows = 16 * seq_len
    grid = (n // rows,)

    def body(h_ref, tt_ref, pos_ref, type_ref, g_ref, b_ref, o_ref):
        x = h_ref[...]
        x = (x.reshape(rows // seq_len, seq_len, DIM) + pos_ref[...][None]
             ).reshape(rows, DIM)
        t0 = type_ref[0:1, :]
        dt = type_ref[1:2, :] - t0
        x = x + t0 + tt_ref[...] * dt
        mean = jnp.mean(x, axis=-1, keepdims=True)
        xc = x - mean
        var = jnp.mean(xc * xc, axis=-1, keepdims=True)
        o_ref[...] = xc * lax.rsqrt(var + EPS) * g_ref[...] + b_ref[...]

    return pl.pallas_call(
        body,
        grid=grid,
        in_specs=[
            pl.BlockSpec((rows, DIM), lambda i: (i, 0)),
            pl.BlockSpec((rows, 1), lambda i: (i, 0)),
            pl.BlockSpec((seq_len, DIM), lambda i: (0, 0)),
            pl.BlockSpec((2, DIM), lambda i: (0, 0)),
            pl.BlockSpec((1, DIM), lambda i: (0, 0)),
            pl.BlockSpec((1, DIM), lambda i: (0, 0)),
        ],
        out_specs=pl.BlockSpec((rows, DIM), lambda i: (i, 0)),
        out_shape=jax.ShapeDtypeStruct((n, DIM), jnp.float32),
    )(h, ttf, pos, type_table, gamma, beta)


def kernel(x, token_type, token_table, pos_table, type_table, gamma, beta):
    b, l = x.shape
    n = b * l
    x3d = x.reshape(NUM_WORKERS, n // (NUM_WORKERS * CHUNK),
                    CHUNK).astype(jnp.int32)
    tt3d = token_type.reshape(x3d.shape).astype(jnp.int32)
    posb = pos_table[:l] + type_table[0][None, :]
    dt = type_table[1] - type_table[0]
    out = _sc_fused(x3d, tt3d, posb, dt, gamma, beta, token_table, l)
    return out.reshape(b, l, DIM)


# speedup vs baseline: 1.0311x; 1.0311x over previous
import functools

import jax
import jax.numpy as jnp
from jax import lax
from jax.experimental import pallas as pl
from jax.experimental.pallas import tpu as pltpu
from jax.experimental.pallas import tpu_sc as plsc

DIM = 128
EPS = 1e-12
NUM_WORKERS = 32
CHUNK = 128


def _sc_token_gather(x2d, token_table):
    n = x2d.shape[0] * x2d.shape[1] * CHUNK
    per_w = n // NUM_WORKERS
    n_chunks = per_w // CHUNK
    mesh = plsc.VectorSubcoreMesh(core_axis_name="c", subcore_axis_name="s")

    @functools.partial(
        pl.kernel,
        mesh=mesh,
        out_type=jax.ShapeDtypeStruct((n, DIM), jnp.float32),
        scratch_types=[
            pltpu.VMEM((n_chunks, CHUNK), jnp.int32),
            pltpu.VMEM((CHUNK, DIM), jnp.float32),
            pltpu.VMEM((CHUNK, DIM), jnp.float32),
            pltpu.SemaphoreType.DMA,
            pltpu.SemaphoreType.DMA,
            pltpu.SemaphoreType.DMA,
            pltpu.SemaphoreType.DMA,
        ],
    )
    def k(x_ref, tab_ref, out_ref, idx_all, r0, r1, gsem0, gsem1, wsem0,
          wsem1):
        num_cores = 2
        wid = lax.axis_index("s") * num_cores + lax.axis_index("c")
        base_w = wid * per_w

        def g_start(c, buf, sem):
            pltpu.async_copy(tab_ref.at[idx_all.at[c]], buf, sem)

        def g_wait(c, buf, sem):
            pltpu.make_async_copy(tab_ref.at[idx_all.at[c]], buf, sem).wait()

        def w_start(c, buf, sem):
            pltpu.async_copy(buf, out_ref.at[pl.ds(base_w + c * CHUNK, CHUNK)],
                             sem)

        def w_wait(c, buf, sem):
            pltpu.make_async_copy(
                buf, out_ref.at[pl.ds(base_w + c * CHUNK, CHUNK)], sem).wait()

        pltpu.sync_copy(x_ref.at[wid], idx_all)
        g_start(0, r0, gsem0)
        g_start(1, r1, gsem1)
        g_wait(0, r0, gsem0)
        w_start(0, r0, wsem0)

        def body(g, carry):
            c = 2 * g
            w_wait(c - 2, r0, wsem0)
            g_start(c, r0, gsem0)
            g_wait(c - 1, r1, gsem1)
            w_start(c - 1, r1, wsem1)
            w_wait(c - 1, r1, wsem1)
            g_start(c + 1, r1, gsem1)
            g_wait(c, r0, gsem0)
            w_start(c, r0, wsem0)
            return carry

        lax.fori_loop(1, n_chunks // 2, body, 0)
        g_wait(n_chunks - 1, r1, gsem1)
        w_start(n_chunks - 1, r1, wsem1)
        w_wait(n_chunks - 2, r0, wsem0)
        w_wait(n_chunks - 1, r1, wsem1)

    return k(x2d, token_table)


def _gather16(vec, idx):
    dnums = lax.GatherDimensionNumbers(
        offset_dims=(), collapsed_slice_dims=(0,), start_index_map=(0,))
    return lax.gather(vec, idx.reshape(16, 1), dnums, (1,),
                      mode=lax.GatherScatterMode.PROMISE_IN_BOUNDS)


def _splat(vec, lane):
    return _gather16(vec, jnp.full((16,), lane, dtype=jnp.int32))


def _rsqrt_newton(x):
    i = lax.bitcast_convert_type(x, jnp.int32).astype(jnp.float32)
    i = (jnp.float32(1597463007.0) - 0.5 * i).astype(jnp.int32)
    y = lax.bitcast_convert_type(i, jnp.float32)
    for _ in range(2):
        y = y * (1.5 - 0.5 * x * y * y)
    return y


def _lanesum(v):
    lanes = lax.iota(jnp.int32, 16)
    for k in (1, 2, 4, 8):
        v = v + _gather16(v, lax.bitwise_xor(lanes, jnp.int32(k)))
    return v


def _sc_fused(x3d, tt3d, posb, dt, gamma, beta, token_table, seq_len):
    n = x3d.shape[0] * x3d.shape[1] * CHUNK
    per_w = n // NUM_WORKERS
    n_chunks = per_w // CHUNK
    n_main = (n_chunks // 3) * 3
    mesh = plsc.VectorSubcoreMesh(core_axis_name="c", subcore_axis_name="s")
    nvec = DIM // 16

    @functools.partial(
        pl.kernel,
        mesh=mesh,
        out_type=jax.ShapeDtypeStruct((n, DIM), jnp.float32),
        scratch_types=[
            pltpu.VMEM((n_chunks, CHUNK), jnp.int32),
            pltpu.VMEM((n_chunks, CHUNK), jnp.int32),
            pltpu.VMEM((seq_len, DIM), jnp.float32),
            pltpu.VMEM((DIM,), jnp.float32),
            pltpu.VMEM((DIM,), jnp.float32),
            pltpu.VMEM((DIM,), jnp.float32),
            pltpu.VMEM((CHUNK, DIM), jnp.float32),
            pltpu.VMEM((CHUNK, DIM), jnp.float32),
            pltpu.VMEM((CHUNK, DIM), jnp.float32),
            pltpu.SemaphoreType.DMA,
            pltpu.SemaphoreType.DMA,
            pltpu.SemaphoreType.DMA,
            pltpu.SemaphoreType.DMA,
            pltpu.SemaphoreType.DMA,
            pltpu.SemaphoreType.DMA,
        ],
    )
    def k(x_ref, tt_ref, posb_ref, dt_ref, g_ref, b_ref, tab_ref, out_ref,
          idx_all, tt_all, posb_v, dt_v, g_v, b_v, r0, r1, r2,
          gs0, gs1, gs2, ws0, ws1, ws2):
        num_cores = 2
        wid = lax.axis_index("s") * num_cores + lax.axis_index("c")
        base_w = wid * per_w
        bufs = (r0, r1, r2)
        gsems = (gs0, gs1, gs2)
        wsems = (ws0, ws1, ws2)

        pltpu.sync_copy(x_ref.at[wid], idx_all)
        pltpu.sync_copy(tt_ref.at[wid], tt_all)
        pltpu.sync_copy(posb_ref, posb_v)
        pltpu.sync_copy(dt_ref, dt_v)
        pltpu.sync_copy(g_ref, g_v)
        pltpu.sync_copy(b_ref, b_v)

        dtv = [dt_v[pl.ds(16 * v, 16)] for v in range(nvec)]
        gv = [g_v[pl.ds(16 * v, 16)] for v in range(nvec)]
        bv = [b_v[pl.ds(16 * v, 16)] for v in range(nvec)]
        inv_d = jnp.float32(1.0 / DIM)

        def g_start(c, buf, sem):
            pltpu.async_copy(tab_ref.at[idx_all.at[c]], buf, sem)

        def g_wait(c, buf, sem):
            pltpu.make_async_copy(tab_ref.at[idx_all.at[c]], buf, sem).wait()

        def w_start(c, buf, sem):
            pltpu.async_copy(buf, out_ref.at[pl.ds(base_w + c * CHUNK, CHUNK)],
                             sem)

        def w_wait(c, buf, sem):
            pltpu.make_async_copy(
                buf, out_ref.at[pl.ds(base_w + c * CHUNK, CHUNK)], sem).wait()

        def compute(c, buf):
            base_l = lax.rem(base_w + c * CHUNK, seq_len)

            @plsc.parallel_loop(0, CHUNK, unroll=4)
            def token(j):
                l0 = base_l + j
                l = jnp.where(l0 >= seq_len, l0 - seq_len, l0)
                ttv = tt_all[c, pl.ds((j // 16) * 16, 16)]
                ttf = _splat(ttv, lax.rem(j, 16)).astype(jnp.float32)
                h = [buf[j, pl.ds(16 * v, 16)]
                     + posb_v[l, pl.ds(16 * v, 16)]
                     + ttf * dtv[v] for v in range(nvec)]
                s = h[0]
                q = h[0] * h[0]
                for v in range(1, nvec):
                    s = s + h[v]
                    q = q + h[v] * h[v]
                mean = _lanesum(s) * inv_d
                var = _lanesum(q) * inv_d - mean * mean
                rstd = _rsqrt_newton(var + EPS)
                mb = mean * rstd
                for v in range(nvec):
                    buf[j, pl.ds(16 * v, 16)] = ((h[v] * rstd - mb) * gv[v]
                                                 + bv[v])

        g_start(0, bufs[0], gsems[0])
        g_start(1, bufs[1], gsems[1])

        def group(i, carry):
            for kk in range(3):
                c = 3 * i + kk
                g_wait(c, bufs[kk], gsems[kk])
                compute(c, bufs[kk])
                if kk == 0:
                    @pl.when(i > 0)
                    def _():
                        w_wait(c - 1, bufs[2], wsems[2])
                else:
                    w_wait(c - 1, bufs[kk - 1], wsems[kk - 1])
                g_start(c + 2, bufs[(kk + 2) % 3], gsems[(kk + 2) % 3])
                w_start(c, bufs[kk], wsems[kk])
            return carry

        lax.fori_loop(0, n_main // 3, group, 0)

        g_wait(n_main, bufs[0], gsems[0])
        compute(n_main, bufs[0])
        w_wait(n_main - 1, bufs[2], wsems[2])
        w_start(n_main, bufs[0], wsems[0])
        g_wait(n_main + 1, bufs[1], gsems[1])
        compute(n_main + 1, bufs[1])
        w_start(n_main + 1, bufs[1], wsems[1])
        w_wait(n_main, bufs[0], wsems[0])
        w_wait(n_main + 1, bufs[1], wsems[1])

    return k(x3d, tt3d, posb, dt, gamma, beta, token_table)


def _tc_ln(h, ttf, pos, type_table, gamma, beta, seq_len):
    n = h.shape[0]
    rows = 16 * seq_len
    grid = (n // rows,)

    def body(h_ref, tt_ref, pos_ref, type_ref, g_ref, b_ref, o_ref):
        x = h_ref[...]
        x = (x.reshape(rows // seq_len, seq_len, DIM) + pos_ref[...][None]
             ).reshape(rows, DIM)
        t0 = type_ref[0:1, :]
        dt = type_ref[1:2, :] - t0
        x = x + t0 + tt_ref[...] * dt
        mean = jnp.mean(x, axis=-1, keepdims=True)
        xc = x - mean
        var = jnp.mean(xc * xc, axis=-1, keepdims=True)
        o_ref[...] = xc * lax.rsqrt(var + EPS) * g_ref[...] + b_ref[...]

    return pl.pallas_call(
        body,
        grid=grid,
        in_specs=[
            pl.BlockSpec((rows, DIM), lambda i: (i, 0)),
            pl.BlockSpec((rows, 1), lambda i: (i, 0)),
            pl.BlockSpec((seq_len, DIM), lambda i: (0, 0)),
            pl.BlockSpec((2, DIM), lambda i: (0, 0)),
            pl.BlockSpec((1, DIM), lambda i: (0, 0)),
            pl.BlockSpec((1, DIM), lambda i: (0, 0)),
        ],
        out_specs=pl.BlockSpec((rows, DIM), lambda i: (i, 0)),
        out_shape=jax.ShapeDtypeStruct((n, DIM), jnp.float32),
    )(h, ttf, pos, type_table, gamma, beta)


def kernel(x, token_type, token_table, pos_table, type_table, gamma, beta):
    b, l = x.shape
    n = b * l
    x3d = x.reshape(NUM_WORKERS, n // (NUM_WORKERS * CHUNK),
                    CHUNK).astype(jnp.int32)
    tt3d = token_type.reshape(x3d.shape).astype(jnp.int32)
    posb = pos_table[:l] + type_table[0][None, :]
    dt = type_table[1] - type_table[0]
    out = _sc_fused(x3d, tt3d, posb, dt, gamma, beta, token_table, l)
    return out.reshape(b, l, DIM)


# speedup vs baseline: 1.0745x; 1.0421x over previous
import functools

import jax
import jax.numpy as jnp
from jax import lax
from jax.experimental import pallas as pl
from jax.experimental.pallas import tpu as pltpu
from jax.experimental.pallas import tpu_sc as plsc

DIM = 128
EPS = 1e-12
NUM_WORKERS = 32
CHUNK = 128


def _gather16(vec, idx):
    dnums = lax.GatherDimensionNumbers(
        offset_dims=(), collapsed_slice_dims=(0,), start_index_map=(0,))
    return lax.gather(vec, idx.reshape(16, 1), dnums, (1,),
                      mode=lax.GatherScatterMode.PROMISE_IN_BOUNDS)


def _splat(vec, lane):
    return _gather16(vec, jnp.full((16,), lane, dtype=jnp.int32))


def _rsqrt_newton(x):
    i = lax.bitcast_convert_type(x, jnp.int32).astype(jnp.float32)
    i = (jnp.float32(1597463007.0) - 0.5 * i).astype(jnp.int32)
    y = lax.bitcast_convert_type(i, jnp.float32)
    for _ in range(3):
        y = y * (1.5 - 0.5 * x * y * y)
    return y


def _lanesum(v):
    lanes = lax.iota(jnp.int32, 16)
    for k in (1, 2, 4, 8):
        v = v + _gather16(v, lax.bitwise_xor(lanes, jnp.int32(k)))
    return v


def _sc_fused(x3d, tt3d, posb, dt, gamma, beta, token_table, seq_len):
    n = x3d.shape[0] * x3d.shape[1] * CHUNK
    per_w = n // NUM_WORKERS
    n_chunks = per_w // CHUNK
    n_main = (n_chunks // 3) * 3
    mesh = plsc.VectorSubcoreMesh(core_axis_name="c", subcore_axis_name="s")
    nvec = DIM // 16

    @functools.partial(
        pl.kernel,
        mesh=mesh,
        out_type=jax.ShapeDtypeStruct((n, DIM), jnp.float32),
        scratch_types=[
            pltpu.VMEM((n_chunks, CHUNK), jnp.int32),
            pltpu.VMEM((n_chunks, CHUNK), jnp.int32),
            pltpu.VMEM((seq_len, DIM), jnp.float32),
            pltpu.VMEM((DIM,), jnp.float32),
            pltpu.VMEM((DIM,), jnp.float32),
            pltpu.VMEM((DIM,), jnp.float32),
            pltpu.VMEM((CHUNK, DIM), jnp.float32),
            pltpu.VMEM((CHUNK, DIM), jnp.float32),
            pltpu.VMEM((CHUNK, DIM), jnp.float32),
            pltpu.SemaphoreType.DMA,
            pltpu.SemaphoreType.DMA,
            pltpu.SemaphoreType.DMA,
            pltpu.SemaphoreType.DMA,
            pltpu.SemaphoreType.DMA,
            pltpu.SemaphoreType.DMA,
        ],
    )
    def k(x_ref, tt_ref, posb_ref, dt_ref, g_ref, b_ref, tab_ref, out_ref,
          idx_all, tt_all, posb_v, dt_v, g_v, b_v, r0, r1, r2,
          gs0, gs1, gs2, ws0, ws1, ws2):
        num_cores = 2
        wid = lax.axis_index("s") * num_cores + lax.axis_index("c")
        base_w = wid * per_w
        bufs = (r0, r1, r2)
        gsems = (gs0, gs1, gs2)
        wsems = (ws0, ws1, ws2)

        pltpu.sync_copy(x_ref.at[wid], idx_all)
        pltpu.sync_copy(tt_ref.at[wid], tt_all)
        pltpu.sync_copy(posb_ref, posb_v)
        pltpu.sync_copy(dt_ref, dt_v)
        pltpu.sync_copy(g_ref, g_v)
        pltpu.sync_copy(b_ref, b_v)

        dtv = [dt_v[pl.ds(16 * v, 16)] for v in range(nvec)]
        gv = [g_v[pl.ds(16 * v, 16)] for v in range(nvec)]
        bv = [b_v[pl.ds(16 * v, 16)] for v in range(nvec)]
        inv_d = jnp.float32(1.0 / DIM)

        def g_start(c, buf, sem):
            pltpu.async_copy(tab_ref.at[idx_all.at[c]], buf, sem)

        def g_wait(c, buf, sem):
            pltpu.make_async_copy(tab_ref.at[idx_all.at[c]], buf, sem).wait()

        def w_start(c, buf, sem):
            pltpu.async_copy(buf, out_ref.at[pl.ds(base_w + c * CHUNK, CHUNK)],
                             sem)

        def w_wait(c, buf, sem):
            pltpu.make_async_copy(
                buf, out_ref.at[pl.ds(base_w + c * CHUNK, CHUNK)], sem).wait()

        def compute(c, buf):
            base_l = lax.rem(base_w + c * CHUNK, seq_len)

            @plsc.parallel_loop(0, CHUNK, unroll=4)
            def token(j):
                l0 = base_l + j
                l = jnp.where(l0 >= seq_len, l0 - seq_len, l0)
                ttv = tt_all[c, pl.ds((j // 16) * 16, 16)]
                ttf = _splat(ttv, lax.rem(j, 16)).astype(jnp.float32)
                h = [buf[j, pl.ds(16 * v, 16)]
                     + posb_v[l, pl.ds(16 * v, 16)]
                     + ttf * dtv[v] for v in range(nvec)]
                s = h[0]
                q = h[0] * h[0]
                for v in range(1, nvec):
                    s = s + h[v]
                    q = q + h[v] * h[v]
                mean = _lanesum(s) * inv_d
                var = _lanesum(q) * inv_d - mean * mean
                rstd = _rsqrt_newton(var + EPS)
                mb = mean * rstd
                for v in range(nvec):
                    buf[j, pl.ds(16 * v, 16)] = ((h[v] * rstd - mb) * gv[v]
                                                 + bv[v])

        g_start(0, bufs[0], gsems[0])
        g_start(1, bufs[1], gsems[1])

        def group(i, carry):
            for kk in range(3):
                c = 3 * i + kk
                g_wait(c, bufs[kk], gsems[kk])
                compute(c, bufs[kk])
                if kk == 0:
                    @pl.when(i > 0)
                    def _():
                        w_wait(c - 1, bufs[2], wsems[2])
                else:
                    w_wait(c - 1, bufs[kk - 1], wsems[kk - 1])
                g_start(c + 2, bufs[(kk + 2) % 3], gsems[(kk + 2) % 3])
                w_start(c, bufs[kk], wsems[kk])
            return carry

        lax.fori_loop(0, n_main // 3, group, 0)

        g_wait(n_main, bufs[0], gsems[0])
        compute(n_main, bufs[0])
        w_wait(n_main - 1, bufs[2], wsems[2])
        w_start(n_main, bufs[0], wsems[0])
        g_wait(n_main + 1, bufs[1], gsems[1])
        compute(n_main + 1, bufs[1])
        w_start(n_main + 1, bufs[1], wsems[1])
        w_wait(n_main, bufs[0], wsems[0])
        w_wait(n_main + 1, bufs[1], wsems[1])

    return k(x3d, tt3d, posb, dt, gamma, beta, token_table)


def kernel(x, token_type, token_table, pos_table, type_table, gamma, beta):
    b, l = x.shape
    n = b * l
    x3d = x.reshape(NUM_WORKERS, n // (NUM_WORKERS * CHUNK),
                    CHUNK).astype(jnp.int32)
    tt3d = token_type.reshape(x3d.shape).astype(jnp.int32)
    posb = pos_table[:l] + type_table[0][None, :]
    dt = type_table[1] - type_table[0]
    out = _sc_fused(x3d, tt3d, posb, dt, gamma, beta, token_table, l)
    return out.reshape(b, l, DIM)
